# Initial kernel scaffold; baseline (speedup 1.0000x reference)
import jax
import jax.numpy as jnp
from jax.experimental import pallas as pl


def kernel(premise, hypothesis, table, W, b):
    raise NotImplementedError("write your pallas kernel here")


# speedup vs baseline: 58.3553x; 58.3553x over previous
import functools
import jax
import jax.numpy as jnp
from jax import lax
from jax.experimental import pallas as pl
from jax.experimental.pallas import tpu as pltpu
from jax.experimental.pallas import tpu_sc as plsc

VOCAB = 1000000
D = 32
B = 4096
S = 200
NC = 2
NS = 16
NW = NC * NS
ROWS_PER_W = B // NW
L = 16


def _body(premise_hbm, hypothesis_hbm, table_hbm, wb_hbm, out_hbm,
          idx_p, idx_h, rows_p, rows_h, wb_v, out_v, sem_p, sem_h):
    wid = lax.axis_index("s") * NC + lax.axis_index("c")
    base = wid * ROWS_PER_W

    pltpu.sync_copy(premise_hbm.at[pl.ds(base, ROWS_PER_W)], idx_p)
    pltpu.sync_copy(hypothesis_hbm.at[pl.ds(base, ROWS_PER_W)], idx_h)
    pltpu.sync_copy(wb_hbm, wb_v)

    w0 = wb_v[pl.ds(0, L)]
    w1 = wb_v[pl.ds(16, L)]
    w2 = wb_v[pl.ds(32, L)]
    w3 = wb_v[pl.ds(48, L)]
    bv = wb_v[pl.ds(64, L)]
    lanes = lax.iota(jnp.int32, L)
    neg = jnp.full((L,), -jnp.inf, jnp.float32)

    def blk_loop(blk, _):
        def row_loop(i, acc):
            r = blk * L + i
            cp_p = pltpu.make_async_copy(table_hbm.at[idx_p.at[r]], rows_p, sem_p)
            cp_h = pltpu.make_async_copy(table_hbm.at[idx_h.at[r]], rows_h, sem_h)
            cp_p.start()
            cp_h.start()
            cp_p.wait()
            cp_h.wait()

            def mx(j, carry):
                m0, m1, m2, m3 = carry
                m0 = jnp.maximum(m0, rows_p[j, pl.ds(0, L)])
                m1 = jnp.maximum(m1, rows_p[j, pl.ds(16, L)])
                m2 = jnp.maximum(m2, rows_h[j, pl.ds(0, L)])
                m3 = jnp.maximum(m3, rows_h[j, pl.ds(16, L)])
                return (m0, m1, m2, m3)

            m0, m1, m2, m3 = lax.fori_loop(0, S, mx, (neg, neg, neg, neg),
                                           unroll=8)
            part = m0 * w0 + m1 * w1 + m2 * w2 + m3 * w3
            dnums = lax.GatherDimensionNumbers(
                offset_dims=(), collapsed_slice_dims=(0,),
                start_index_map=(0,))
            for off in (8, 4, 2, 1):
                perm = lax.gather(
                    part, (lanes ^ off)[:, None], dnums, (1,),
                    mode=lax.GatherScatterMode.PROMISE_IN_BOUNDS)
                part = part + perm
            return jnp.where(lanes == i, part, acc)

        acc = lax.fori_loop(0, L, row_loop, jnp.zeros((L,), jnp.float32))
        prob = 1.0 / (1.0 + jnp.exp(-(acc + bv)))
        out_v[pl.ds(blk * L, L)] = prob
        return 0

    lax.fori_loop(0, ROWS_PER_W // L, blk_loop, 0)
    pltpu.sync_copy(out_v, out_hbm.at[pl.ds(base, ROWS_PER_W)])


@jax.jit
def _run(premise, hypothesis, table, wb):
    mesh = plsc.VectorSubcoreMesh(core_axis_name="c", subcore_axis_name="s")
    f = functools.partial(
        pl.kernel,
        mesh=mesh,
        out_type=jax.ShapeDtypeStruct((B,), jnp.float32),
        compiler_params=pltpu.CompilerParams(use_tc_tiling_on_sc=False),
        scratch_types=[
            pltpu.VMEM((ROWS_PER_W, S), jnp.int32),
            pltpu.VMEM((ROWS_PER_W, S), jnp.int32),
            pltpu.VMEM((S, D), jnp.float32),
            pltpu.VMEM((S, D), jnp.float32),
            pltpu.VMEM((80,), jnp.float32),
            pltpu.VMEM((ROWS_PER_W,), jnp.float32),
            pltpu.SemaphoreType.DMA,
            pltpu.SemaphoreType.DMA,
        ],
    )(_body)
    return f(premise, hypothesis, table, wb)


def kernel(premise, hypothesis, table, W, b):
    premise = premise.astype(jnp.int32)
    hypothesis = hypothesis.astype(jnp.int32)
    wb = jnp.concatenate(
        [W.reshape(2 * D).astype(jnp.float32),
         jnp.broadcast_to(b.astype(jnp.float32), (L,))])
    return _run(premise, hypothesis, table, wb)


# speedup vs baseline: 66.0251x; 1.1314x over previous
import functools
import jax
import jax.numpy as jnp
from jax import lax
from jax.experimental import pallas as pl
from jax.experimental.pallas import tpu as pltpu
from jax.experimental.pallas import tpu_sc as plsc

VOCAB = 1000000
D = 32
B = 4096
S = 200
NC = 2
NS = 16
NW = NC * NS
ROWS_PER_W = B // NW
L = 16


def _body(premise_hbm, hypothesis_hbm, table_hbm, wb_hbm, out_hbm,
          idx_p, idx_h, rows_p0, rows_h0, rows_p1, rows_h1,
          wb_v, out_v, sem0, sem1):
    wid = lax.axis_index("s") * NC + lax.axis_index("c")
    base = wid * ROWS_PER_W

    pltpu.sync_copy(premise_hbm.at[pl.ds(base, ROWS_PER_W)], idx_p)
    pltpu.sync_copy(hypothesis_hbm.at[pl.ds(base, ROWS_PER_W)], idx_h)
    pltpu.sync_copy(wb_hbm, wb_v)

    w0 = wb_v[pl.ds(0, L)]
    w1 = wb_v[pl.ds(16, L)]
    w2 = wb_v[pl.ds(32, L)]
    w3 = wb_v[pl.ds(48, L)]
    bv = wb_v[pl.ds(64, L)]
    lanes = lax.iota(jnp.int32, L)
    neg = jnp.full((L,), -jnp.inf, jnp.float32)
    dnums = lax.GatherDimensionNumbers(
        offset_dims=(), collapsed_slice_dims=(0,), start_index_map=(0,))

    def start_pair(r, rows_pb, rows_hb, sem):
        pltpu.make_async_copy(table_hbm.at[idx_p.at[r]], rows_pb, sem).start()
        pltpu.make_async_copy(table_hbm.at[idx_h.at[r]], rows_hb, sem).start()

    def wait_pair(rows_pb, rows_hb, sem):
        pltpu.make_async_copy(table_hbm.at[idx_p.at[0]], rows_pb, sem).wait()
        pltpu.make_async_copy(table_hbm.at[idx_h.at[0]], rows_hb, sem).wait()

    def compute_row(rows_pb, rows_hb):
        def mx(j, carry):
            m0, m1, m2, m3 = carry
            m0 = jnp.maximum(m0, rows_pb[j, pl.ds(0, L)])
            m1 = jnp.maximum(m1, rows_pb[j, pl.ds(16, L)])
            m2 = jnp.maximum(m2, rows_hb[j, pl.ds(0, L)])
            m3 = jnp.maximum(m3, rows_hb[j, pl.ds(16, L)])
            return (m0, m1, m2, m3)

        m0, m1, m2, m3 = lax.fori_loop(0, S, mx, (neg, neg, neg, neg),
                                       unroll=8)
        part = m0 * w0 + m1 * w1 + m2 * w2 + m3 * w3
        for off in (8, 4, 2, 1):
            perm = lax.gather(
                part, (lanes ^ off)[:, None], dnums, (1,),
                mode=lax.GatherScatterMode.PROMISE_IN_BOUNDS)
            part = part + perm
        return part

    start_pair(0, rows_p0, rows_h0, sem0)

    def body2(g, acc):
        r0 = 2 * g
        r1 = r0 + 1
        start_pair(r1, rows_p1, rows_h1, sem1)
        wait_pair(rows_p0, rows_h0, sem0)
        v = compute_row(rows_p0, rows_h0)
        acc = jnp.where(lanes == (r0 & 15), v, acc)
        start_pair(jnp.minimum(r1 + 1, ROWS_PER_W - 1), rows_p0, rows_h0,
                   sem0)
        wait_pair(rows_p1, rows_h1, sem1)
        v = compute_row(rows_p1, rows_h1)
        acc = jnp.where(lanes == (r1 & 15), v, acc)

        @pl.when((r1 & 15) == 15)
        def _flush():
            out_v[pl.ds((r1 >> 4) * L, L)] = 1.0 / (1.0 + jnp.exp(-(acc + bv)))

        return acc

    lax.fori_loop(0, ROWS_PER_W // 2, body2, jnp.zeros((L,), jnp.float32))
    wait_pair(rows_p0, rows_h0, sem0)
    pltpu.sync_copy(out_v, out_hbm.at[pl.ds(base, ROWS_PER_W)])


@jax.jit
def _run(premise, hypothesis, table, wb):
    mesh = plsc.VectorSubcoreMesh(core_axis_name="c", subcore_axis_name="s")
    f = functools.partial(
        pl.kernel,
        mesh=mesh,
        out_type=jax.ShapeDtypeStruct((B,), jnp.float32),
        compiler_params=pltpu.CompilerParams(use_tc_tiling_on_sc=False),
        scratch_types=[
            pltpu.VMEM((ROWS_PER_W, S), jnp.int32),
            pltpu.VMEM((ROWS_PER_W, S), jnp.int32),
            pltpu.VMEM((S, D), jnp.float32),
            pltpu.VMEM((S, D), jnp.float32),
            pltpu.VMEM((S, D), jnp.float32),
            pltpu.VMEM((S, D), jnp.float32),
            pltpu.VMEM((80,), jnp.float32),
            pltpu.VMEM((ROWS_PER_W,), jnp.float32),
            pltpu.SemaphoreType.DMA,
            pltpu.SemaphoreType.DMA,
        ],
    )(_body)
    return f(premise, hypothesis, table, wb)


def kernel(premise, hypothesis, table, W, b):
    premise = premise.astype(jnp.int32)
    hypothesis = hypothesis.astype(jnp.int32)
    wb = jnp.concatenate(
        [W.reshape(2 * D).astype(jnp.float32),
         jnp.broadcast_to(b.astype(jnp.float32), (L,))])
    return _run(premise, hypothesis, table, wb)
